# baseline (speedup 1.0000x reference)
import jax
import jax.numpy as jnp
import numpy as np
from jax.experimental import pallas as pl

N = 10000
E = 160000
D_IN = 256
HID = 512
HEADS = 8
C = HID // HEADS
NQ = 200
AH = 8
AC = D_IN // AH
NEG_SLOPE = 0.2


def _gat_layer(h, p, src, dst):
    n = h.shape[0]
    hp = (h @ p['W']).reshape(n, HEADS, C)
    e_src = jnp.sum(hp * p['a_src'], axis=-1)
    e_dst = jnp.sum(hp * p['a_dst'], axis=-1)
    e = e_src[src] + e_dst[dst]
    e = jnp.where(e > 0, e, NEG_SLOPE * e)
    m = jax.ops.segment_max(e, dst, num_segments=n)
    ex = jnp.exp(e - m[dst])
    den = jax.ops.segment_sum(ex, dst, num_segments=n)
    alpha = ex / (den[dst] + 1e-16)
    msg = hp[src] * alpha[:, :, None]
    out = jax.ops.segment_sum(msg, dst, num_segments=n).reshape(n, HEADS * C)
    return out + p['b']


def _ln_kernel(x_ref, g_ref, b_ref, o_ref):
    x = x_ref[...]
    m = jnp.mean(x, axis=-1, keepdims=True)
    v = jnp.mean((x - m) ** 2, axis=-1, keepdims=True)
    o_ref[...] = (x - m) / jnp.sqrt(v + 1e-5) * g_ref[...] + b_ref[...]


def _pallas_ln(x, g, b):
    return pl.pallas_call(
        _ln_kernel,
        out_shape=jax.ShapeDtypeStruct(x.shape, x.dtype),
    )(x, g[None, :], b[None, :])


def kernel(x, params, edge_index):
    src = edge_index[0]
    dst = edge_index[1]
    h = x
    ngat = len(params['gat'])
    for i, p in enumerate(params['gat']):
        h = _gat_layer(h, p, src, dst)
        if i < ngat - 1:
            h = jax.nn.elu(h)
    g = jax.nn.gelu(h @ params['proj_W1'] + params['proj_b1']) @ params['proj_W2'] + params['proj_b2']
    go = g[None, :, :]
    q = params['queries'][None, :, :]
    Q = (q @ params['Wq'] + params['bq']).reshape(1, NQ, AH, AC).transpose(0, 2, 1, 3)
    K = (go @ params['Wk'] + params['bk']).reshape(1, N, AH, AC).transpose(0, 2, 1, 3)
    V = (go @ params['Wv'] + params['bv']).reshape(1, N, AH, AC).transpose(0, 2, 1, 3)
    scores = jnp.einsum('bhqd,bhkd->bhqk', Q, K) / float(np.sqrt(AC))
    attn = jax.nn.softmax(scores, axis=-1)
    ctx = jnp.einsum('bhqk,bhkd->bhqd', attn, V).transpose(0, 2, 1, 3).reshape(1, NQ, D_IN)
    ao = ctx @ params['Wo'] + params['bo']
    h1 = _pallas_ln((q + ao)[0], params['ln1_g'], params['ln1_b'])[None]
    f = jax.nn.gelu(h1 @ params['ffn_W1'] + params['ffn_b1']) @ params['ffn_W2'] + params['ffn_b2']
    return _pallas_ln((h1 + f)[0], params['ln2_g'], params['ln2_b'])[None]


# speedup vs baseline: 5.3661x; 5.3661x over previous
import functools

import jax
import jax.numpy as jnp
import numpy as np
from jax import lax
from jax.experimental import pallas as pl
from jax.experimental.pallas import tpu as pltpu
from jax.experimental.pallas import tpu_sc as plsc

N = 10000
E = 160000
D_IN = 256
HID = 512
HEADS = 8
C = HID // HEADS
NQ = 200
AH = 8
AC = D_IN // AH
NEG_SLOPE = 0.2

NC = 2
NS = 16
LANES = 16
NW = NC * NS

NPAD = 12544
SENT = 10240
EPAD = 163840
ESH = EPAD // NW
GRP = ESH // LANES
STRIPE = NPAD // NS

_mesh = plsc.VectorSubcoreMesh(core_axis_name="c", subcore_axis_name="s")
_sc_params = pltpu.CompilerParams(needs_layout_passes=False)


@functools.partial(
    pl.kernel,
    out_type=[
        jax.ShapeDtypeStruct((NC * HEADS * NPAD,), jnp.float32),
        jax.ShapeDtypeStruct((HEADS * EPAD,), jnp.float32),
    ],
    mesh=_mesh,
    compiler_params=_sc_params,
    scratch_types=[
        pltpu.VMEM((ESH,), jnp.int32),
        pltpu.VMEM((ESH,), jnp.int32),
        pltpu.VMEM((NPAD,), jnp.float32),
        pltpu.VMEM((NPAD,), jnp.float32),
        pltpu.VMEM((NPAD,), jnp.float32),
        pltpu.VMEM((ESH,), jnp.float32),
        pltpu.VMEM((STRIPE,), jnp.float32),
        pltpu.VMEM((STRIPE,), jnp.float32),
        pltpu.VMEM_SHARED((NS * NPAD,), jnp.float32),
    ],
)
def _sc_den(es_hbm, ed_hbm, src_hbm, dst_hbm, den_hbm, ex_hbm,
            src_v, dst_v, es_v, ed_v, den_v, ex_v, tmp_v, acc_v, slots):
    c = lax.axis_index("c")
    s = lax.axis_index("s")
    w = 2 * s + c
    e0 = w * ESH
    pltpu.sync_copy(src_hbm.at[pl.ds(e0, ESH)], src_v)
    pltpu.sync_copy(dst_hbm.at[pl.ds(e0, ESH)], dst_v)
    zero16 = jnp.zeros((LANES,), jnp.float32)
    for h in range(HEADS):
        pltpu.sync_copy(es_hbm.at[pl.ds(h * NPAD, NPAD)], es_v)
        pltpu.sync_copy(ed_hbm.at[pl.ds(h * NPAD, NPAD)], ed_v)

        def _zero(i, _):
            den_v[pl.ds(i * LANES, LANES)] = zero16
            return 0

        lax.fori_loop(0, NPAD // LANES, _zero, 0)

        def _edge(i, _):
            si = src_v[pl.ds(i * LANES, LANES)]
            di = dst_v[pl.ds(i * LANES, LANES)]
            a = plsc.load_gather(es_v, [si])
            b = plsc.load_gather(ed_v, [di])
            e = a + b
            e = jnp.where(e > 0, e, NEG_SLOPE * e)
            ex = jnp.exp(e)
            ex_v[pl.ds(i * LANES, LANES)] = ex
            plsc.addupdate_scatter(den_v, [di], ex)
            return 0

        lax.fori_loop(0, GRP, _edge, 0)
        pltpu.sync_copy(ex_v, ex_hbm.at[pl.ds(h * EPAD + e0, ESH)])

        pltpu.sync_copy(den_v, slots.at[pl.ds(s * NPAD, NPAD)])
        plsc.subcore_barrier()
        base = s * STRIPE

        def _zs(i, _):
            acc_v[pl.ds(i * LANES, LANES)] = zero16
            return 0

        lax.fori_loop(0, STRIPE // LANES, _zs, 0)
        for j in range(NS):
            pltpu.sync_copy(slots.at[pl.ds(j * NPAD + base, STRIPE)], tmp_v)

            def _acc(i, _):
                sl = pl.ds(i * LANES, LANES)
                acc_v[sl] = acc_v[sl] + tmp_v[sl]
                return 0

            lax.fori_loop(0, STRIPE // LANES, _acc, 0)
        pltpu.sync_copy(acc_v, den_hbm.at[pl.ds((c * HEADS + h) * NPAD + base, STRIPE)])
        plsc.subcore_barrier()


W = 96
NRND = 4
NOUT = NRND * NW * W
GBLK = 1024
NBLK = EPAD // GBLK
SEL = GBLK + 64
FB = 64
ACCW = (W + 8) * HID


@functools.partial(
    pl.kernel,
    out_type=jax.ShapeDtypeStruct((NOUT * HID,), jnp.float32),
    mesh=_mesh,
    compiler_params=_sc_params,
    scratch_types=[
        pltpu.VMEM((GBLK,), jnp.int32),
        pltpu.VMEM((GBLK,), jnp.int32),
        pltpu.VMEM((SEL,), jnp.int32),
        pltpu.VMEM((SEL,), jnp.int32),
        pltpu.VMEM((SEL,), jnp.int32),
        pltpu.VMEM((FB,), jnp.int32),
        pltpu.VMEM((HEADS * FB,), jnp.float32),
        pltpu.VMEM((HEADS * FB,), jnp.float32),
        pltpu.VMEM((HEADS * W,), jnp.float32),
        pltpu.VMEM((W,), jnp.float32),
        pltpu.VMEM((W,), jnp.float32),
        pltpu.VMEM((FB, HID), jnp.float32),
        pltpu.VMEM((ACCW,), jnp.float32),
    ],
)
def _sc_agg(hp_hbm, ex_hbm, den_hbm, src_hbm, dst_hbm, out_hbm,
            srcg_v, dstg_v, srcsel_v, dlocsel_v, eidsel_v, idxh_v,
            exb_v, alpha_v, invd_v, da_v, db_v, rows_v, acc_v):
    c = lax.axis_index("c")
    s = lax.axis_index("s")
    t = 2 * s + c
    zero16 = jnp.zeros((LANES,), jnp.float32)
    iota16 = lax.iota(jnp.int32, LANES)

    def _zi(i, _):
        sl = pl.ds(i * LANES, LANES)
        z = jnp.zeros((LANES,), jnp.int32)
        srcsel_v[sl] = z
        dlocsel_v[sl] = z
        eidsel_v[sl] = z
        return 0

    lax.fori_loop(0, SEL // LANES, _zi, 0)

    def _flush(q, vc):
        for h in range(HEADS):
            def _ix(i, _):
                sl = pl.ds(i * LANES, LANES)
                idxh_v[sl] = eidsel_v[pl.ds(q * FB + i * LANES, LANES)] + h * EPAD
                return 0

            lax.fori_loop(0, FB // LANES, _ix, 0)
            pltpu.sync_copy(ex_hbm.at[idxh_v], exb_v.at[pl.ds(h * FB, FB)])
        def _al(i, _):
            lane = iota16 + i * LANES
            valid = lane < vc
            d16 = dlocsel_v[pl.ds(q * FB + i * LANES, LANES)]
            gidx = jnp.where(valid, d16, 0)
            for h in range(HEADS):
                ivd = plsc.load_gather(invd_v, [gidx + h * W])
                exv = exb_v[pl.ds(h * FB + i * LANES, LANES)]
                alpha_v[pl.ds(h * FB + i * LANES, LANES)] = \
                    jnp.where(valid, exv * ivd, 0.0)
            return 0

        lax.fori_loop(0, FB // LANES, _al, 0)
        pltpu.sync_copy(hp_hbm.at[srcsel_v.at[pl.ds(q * FB, FB)]], rows_v)

        def _edge(e, _):
            db = plsc.load_gather(dlocsel_v, [jnp.broadcast_to(q * FB + e, (LANES,))])
            addr0 = db * HID + iota16
            for h in range(HEADS):
                av = plsc.load_gather(alpha_v, [jnp.broadcast_to(h * FB + e, (LANES,))])
                for qq in range(4):
                    off = h * C + qq * LANES
                    chunk = rows_v[e, pl.ds(off, LANES)] * av
                    plsc.addupdate_scatter(acc_v, [addr0 + off], chunk)
            return 0

        lax.fori_loop(0, FB, _edge, 0)

    for r in range(NRND):
        lo = (r * NW + t) * W
        for h in range(HEADS):
            pltpu.sync_copy(den_hbm.at[pl.ds(h * NPAD + lo, W)], da_v)
            pltpu.sync_copy(den_hbm.at[pl.ds(HEADS * NPAD + h * NPAD + lo, W)], db_v)

            def _inv(i, _):
                sl = pl.ds(i * LANES, LANES)
                invd_v[pl.ds(h * W + i * LANES, LANES)] = \
                    1.0 / (da_v[sl] + db_v[sl] + 1e-16)
                return 0

            lax.fori_loop(0, W // LANES, _inv, 0)

        def _za(i, _):
            acc_v[pl.ds(i * LANES, LANES)] = zero16
            return 0

        lax.fori_loop(0, ACCW // LANES, _za, 0)

        def _blk(bi, cnt):
            off = bi * GBLK
            pltpu.sync_copy(src_hbm.at[pl.ds(off, GBLK)], srcg_v)
            pltpu.sync_copy(dst_hbm.at[pl.ds(off, GBLK)], dstg_v)

            def _grp(i, cn):
                d16 = dstg_v[pl.ds(i * LANES, LANES)]
                in_w = (d16 >= lo) & (d16 < lo + W) & (d16 < N)
                pc = plsc.all_reduce_population_count(in_w)[0]

                def _stage(cn2):
                    s16 = srcg_v[pl.ds(i * LANES, LANES)]
                    eid = off + i * LANES + iota16
                    plsc.store_compressed(srcsel_v.at[pl.ds(cn2, LANES)], s16, mask=in_w)
                    plsc.store_compressed(dlocsel_v.at[pl.ds(cn2, LANES)], d16 - lo, mask=in_w)
                    plsc.store_compressed(eidsel_v.at[pl.ds(cn2, LANES)], eid, mask=in_w)
                    return cn2 + pc

                return lax.cond(pc > 0, _stage, lambda x: x, cn)

            cnt = lax.fori_loop(0, GBLK // LANES, _grp, cnt)
            nb = cnt // FB

            def _fl(q, _):
                _flush(q, jnp.int32(FB))
                return 0

            lax.fori_loop(0, nb, _fl, 0)

            def _cd():
                rem0 = nb * FB

                def _mv(i, _):
                    sl_src = pl.ds(rem0 + i * LANES, LANES)
                    sl_dst = pl.ds(i * LANES, LANES)
                    srcsel_v[sl_dst] = srcsel_v[sl_src]
                    dlocsel_v[sl_dst] = dlocsel_v[sl_src]
                    eidsel_v[sl_dst] = eidsel_v[sl_src]
                    return 0

                lax.fori_loop(0, FB // LANES, _mv, 0)

            pl.when(nb > 0)(_cd)
            return cnt - nb * FB

        cnt = lax.fori_loop(0, NBLK, _blk, jnp.int32(0))

        def _tail():
            _flush(jnp.int32(0), cnt)

        pl.when(cnt > 0)(_tail)

        pltpu.sync_copy(acc_v.at[pl.ds(0, W * HID)],
                        out_hbm.at[pl.ds(lo * HID, W * HID)])


def _prep_edges(edge_index):
    src = edge_index[0]
    dst = edge_index[1]
    pad = EPAD - E
    src_p = jnp.concatenate([src, jnp.zeros((pad,), jnp.int32)])
    dst_p = jnp.concatenate([dst, jnp.full((pad,), SENT, jnp.int32)])
    return src_p, dst_p


def _gat_layer_a(h, p, src, dst, src_p, dst_p):
    n = h.shape[0]
    hp = (h @ p['W']).reshape(n, HEADS, C)
    e_src = jnp.sum(hp * p['a_src'], axis=-1)
    e_dst = jnp.sum(hp * p['a_dst'], axis=-1)
    es_T = jnp.zeros((HEADS, NPAD), jnp.float32).at[:, :N].set(e_src.T).reshape(-1)
    ed_T = jnp.zeros((HEADS, NPAD), jnp.float32).at[:, :N].set(e_dst.T).reshape(-1)
    den_parts, ex_T = _sc_den(es_T, ed_T, src_p, dst_p)
    out = _sc_agg(hp.reshape(n, HID), ex_T, den_parts, src_p, dst_p)
    return out.reshape(NOUT, HID)[:N] + p['b']


def kernel(x, params, edge_index):
    src = edge_index[0]
    dst = edge_index[1]
    src_p, dst_p = _prep_edges(edge_index)
    h = x
    ngat = len(params['gat'])
    for i, p in enumerate(params['gat']):
        h = _gat_layer_a(h, p, src, dst, src_p, dst_p)
        if i < ngat - 1:
            h = jax.nn.elu(h)
    g = jax.nn.gelu(h @ params['proj_W1'] + params['proj_b1']) @ params['proj_W2'] + params['proj_b2']
    go = g[None, :, :]
    q = params['queries'][None, :, :]
    Q = (q @ params['Wq'] + params['bq']).reshape(1, NQ, AH, AC).transpose(0, 2, 1, 3)
    K = (go @ params['Wk'] + params['bk']).reshape(1, N, AH, AC).transpose(0, 2, 1, 3)
    V = (go @ params['Wv'] + params['bv']).reshape(1, N, AH, AC).transpose(0, 2, 1, 3)
    scores = jnp.einsum('bhqd,bhkd->bhqk', Q, K) / float(np.sqrt(AC))
    attn = jax.nn.softmax(scores, axis=-1)
    ctx = jnp.einsum('bhqk,bhkd->bhqd', attn, V).transpose(0, 2, 1, 3).reshape(1, NQ, D_IN)
    ao = ctx @ params['Wo'] + params['bo']

    def _ln(t, g_, b_):
        m = jnp.mean(t, axis=-1, keepdims=True)
        v = jnp.var(t, axis=-1, keepdims=True)
        return (t - m) / jnp.sqrt(v + 1e-5) * g_ + b_

    h1 = _ln(q + ao, params['ln1_g'], params['ln1_b'])
    f = jax.nn.gelu(h1 @ params['ffn_W1'] + params['ffn_b1']) @ params['ffn_W2'] + params['ffn_b2']
    return _ln(h1 + f, params['ln2_g'], params['ln2_b'])


# speedup vs baseline: 9.1995x; 1.7144x over previous
import functools

import jax
import jax.numpy as jnp
import numpy as np
from jax import lax
from jax.experimental import pallas as pl
from jax.experimental.pallas import tpu as pltpu
from jax.experimental.pallas import tpu_sc as plsc

N = 10000
E = 160000
D_IN = 256
HID = 512
HEADS = 8
C = HID // HEADS
NQ = 200
AH = 8
AC = D_IN // AH
NEG_SLOPE = 0.2

NC = 2
NS = 16
LANES = 16
NW = NC * NS

NPAD = 12544
SENT = 10240
EPAD = 163840
ESH = EPAD // NW
GRP = ESH // LANES
STRIPE = NPAD // NS

_mesh = plsc.VectorSubcoreMesh(core_axis_name="c", subcore_axis_name="s")
_sc_params = pltpu.CompilerParams(needs_layout_passes=False)


@functools.partial(
    pl.kernel,
    out_type=[
        jax.ShapeDtypeStruct((NC * HEADS * NPAD,), jnp.float32),
        jax.ShapeDtypeStruct((HEADS * EPAD,), jnp.float32),
    ],
    mesh=_mesh,
    compiler_params=_sc_params,
    scratch_types=[
        pltpu.VMEM((ESH,), jnp.int32),
        pltpu.VMEM((ESH,), jnp.int32),
        pltpu.VMEM((NPAD,), jnp.float32),
        pltpu.VMEM((NPAD,), jnp.float32),
        pltpu.VMEM((NPAD,), jnp.float32),
        pltpu.VMEM((ESH,), jnp.float32),
        pltpu.VMEM((STRIPE,), jnp.float32),
        pltpu.VMEM((STRIPE,), jnp.float32),
        pltpu.VMEM_SHARED((NS * NPAD,), jnp.float32),
    ],
)
def _sc_den(es_hbm, ed_hbm, src_hbm, dst_hbm, den_hbm, ex_hbm,
            src_v, dst_v, es_v, ed_v, den_v, ex_v, tmp_v, acc_v, slots):
    c = lax.axis_index("c")
    s = lax.axis_index("s")
    w = 2 * s + c
    e0 = w * ESH
    pltpu.sync_copy(src_hbm.at[pl.ds(e0, ESH)], src_v)
    pltpu.sync_copy(dst_hbm.at[pl.ds(e0, ESH)], dst_v)
    zero16 = jnp.zeros((LANES,), jnp.float32)
    for h in range(HEADS):
        pltpu.sync_copy(es_hbm.at[pl.ds(h * NPAD, NPAD)], es_v)
        pltpu.sync_copy(ed_hbm.at[pl.ds(h * NPAD, NPAD)], ed_v)

        def _zero(i, _):
            den_v[pl.ds(i * LANES, LANES)] = zero16
            return 0

        lax.fori_loop(0, NPAD // LANES, _zero, 0)

        def _edge(i, _):
            si = src_v[pl.ds(i * LANES, LANES)]
            di = dst_v[pl.ds(i * LANES, LANES)]
            a = plsc.load_gather(es_v, [si])
            b = plsc.load_gather(ed_v, [di])
            e = a + b
            e = jnp.where(e > 0, e, NEG_SLOPE * e)
            ex = jnp.exp(e)
            ex_v[pl.ds(i * LANES, LANES)] = ex
            plsc.addupdate_scatter(den_v, [di], ex)
            return 0

        lax.fori_loop(0, GRP, _edge, 0)
        pltpu.sync_copy(ex_v, ex_hbm.at[pl.ds(h * EPAD + e0, ESH)])

        pltpu.sync_copy(den_v, slots.at[pl.ds(s * NPAD, NPAD)])
        plsc.subcore_barrier()
        base = s * STRIPE

        def _zs(i, _):
            acc_v[pl.ds(i * LANES, LANES)] = zero16
            return 0

        lax.fori_loop(0, STRIPE // LANES, _zs, 0)
        for j in range(NS):
            pltpu.sync_copy(slots.at[pl.ds(j * NPAD + base, STRIPE)], tmp_v)

            def _acc(i, _):
                sl = pl.ds(i * LANES, LANES)
                acc_v[sl] = acc_v[sl] + tmp_v[sl]
                return 0

            lax.fori_loop(0, STRIPE // LANES, _acc, 0)
        pltpu.sync_copy(acc_v, den_hbm.at[pl.ds((c * HEADS + h) * NPAD + base, STRIPE)])
        plsc.subcore_barrier()


W = 160
NRND = 2
NOUT = NRND * NW * W
GBLK = 2048
NBLK = EPAD // GBLK
SEL = GBLK + 32
FB = 32
ACCW = W * HID


@functools.partial(
    pl.kernel,
    out_type=jax.ShapeDtypeStruct((NOUT * HID,), jnp.float32),
    mesh=_mesh,
    compiler_params=_sc_params,
    scratch_types=[
        pltpu.VMEM((GBLK,), jnp.int32),
        pltpu.VMEM((GBLK,), jnp.int32),
        pltpu.VMEM((SEL,), jnp.int32),
        pltpu.VMEM((SEL,), jnp.int32),
        pltpu.VMEM((SEL,), jnp.int32),
        pltpu.VMEM((HEADS * FB,), jnp.int32),
        pltpu.VMEM((HEADS * FB,), jnp.float32),
        pltpu.VMEM((HEADS * FB,), jnp.float32),
        pltpu.VMEM((HEADS * W,), jnp.float32),
        pltpu.VMEM((W,), jnp.float32),
        pltpu.VMEM((W,), jnp.float32),
        pltpu.VMEM((FB, HID), jnp.float32),
        pltpu.VMEM((ACCW,), jnp.float32),
    ],
)
def _sc_agg(hp_hbm, ex_hbm, den_hbm, src_hbm, dst_hbm, out_hbm,
            srcg_v, dstg_v, srcsel_v, dlocsel_v, eidsel_v, idxh_v,
            exb_v, alpha_v, invd_v, da_v, db_v, rows_v, acc_v):
    c = lax.axis_index("c")
    s = lax.axis_index("s")
    t = 2 * s + c
    zero16 = jnp.zeros((LANES,), jnp.float32)
    iota16 = lax.iota(jnp.int32, LANES)

    def _zi(i, _):
        sl = pl.ds(i * LANES, LANES)
        z = jnp.zeros((LANES,), jnp.int32)
        srcsel_v[sl] = z
        dlocsel_v[sl] = z
        eidsel_v[sl] = z
        return 0

    lax.fori_loop(0, SEL // LANES, _zi, 0)

    def _flush(q, vc):
        def _ix(i, _):
            e16 = eidsel_v[pl.ds(q * FB + i * LANES, LANES)]
            for h in range(HEADS):
                idxh_v[pl.ds(h * FB + i * LANES, LANES)] = e16 + h * EPAD
            return 0

        lax.fori_loop(0, FB // LANES, _ix, 0)
        pltpu.sync_copy(ex_hbm.at[idxh_v], exb_v)
        def _al(i, _):
            lane = iota16 + i * LANES
            valid = lane < vc
            d16 = dlocsel_v[pl.ds(q * FB + i * LANES, LANES)]
            gidx = jnp.where(valid, d16, 0)
            for h in range(HEADS):
                ivd = plsc.load_gather(invd_v, [gidx + h * W])
                exv = exb_v[pl.ds(h * FB + i * LANES, LANES)]
                alpha_v[pl.ds(h * FB + i * LANES, LANES)] = \
                    jnp.where(valid, exv * ivd, 0.0)
            return 0

        lax.fori_loop(0, FB // LANES, _al, 0)
        pltpu.sync_copy(hp_hbm.at[srcsel_v.at[pl.ds(q * FB, FB)]], rows_v)

        def _edge(e, _):
            db = plsc.load_gather(dlocsel_v, [jnp.broadcast_to(q * FB + e, (LANES,))])
            addr0 = db * HID + iota16
            for h in range(HEADS):
                av = plsc.load_gather(alpha_v, [jnp.broadcast_to(h * FB + e, (LANES,))])
                for qq in range(4):
                    off = h * C + qq * LANES
                    chunk = rows_v[e, pl.ds(off, LANES)] * av
                    plsc.addupdate_scatter(acc_v, [addr0 + off], chunk)
            return 0

        lax.fori_loop(0, FB, _edge, 0)

    for r in range(NRND):
        lo = (r * NW + t) * W
        for h in range(HEADS):
            pltpu.sync_copy(den_hbm.at[pl.ds(h * NPAD + lo, W)], da_v)
            pltpu.sync_copy(den_hbm.at[pl.ds(HEADS * NPAD + h * NPAD + lo, W)], db_v)

            def _inv(i, _):
                sl = pl.ds(i * LANES, LANES)
                invd_v[pl.ds(h * W + i * LANES, LANES)] = \
                    1.0 / (da_v[sl] + db_v[sl] + 1e-16)
                return 0

            lax.fori_loop(0, W // LANES, _inv, 0)

        def _za(i, _):
            acc_v[pl.ds(i * LANES, LANES)] = zero16
            return 0

        lax.fori_loop(0, ACCW // LANES, _za, 0)

        def _blk(bi, cnt):
            off = bi * GBLK
            pltpu.sync_copy(src_hbm.at[pl.ds(off, GBLK)], srcg_v)
            pltpu.sync_copy(dst_hbm.at[pl.ds(off, GBLK)], dstg_v)

            def _grp(i, cn):
                d16 = dstg_v[pl.ds(i * LANES, LANES)]
                in_w = (d16 >= lo) & (d16 < lo + W)
                pc = plsc.all_reduce_population_count(in_w)[0]

                def _stage(cn2):
                    s16 = srcg_v[pl.ds(i * LANES, LANES)]
                    eid = off + i * LANES + iota16
                    plsc.store_compressed(srcsel_v.at[pl.ds(cn2, LANES)], s16, mask=in_w)
                    plsc.store_compressed(dlocsel_v.at[pl.ds(cn2, LANES)], d16 - lo, mask=in_w)
                    plsc.store_compressed(eidsel_v.at[pl.ds(cn2, LANES)], eid, mask=in_w)
                    return cn2 + pc

                return lax.cond(pc > 0, _stage, lambda x: x, cn)

            cnt = lax.fori_loop(0, GBLK // LANES, _grp, cnt)
            nb = cnt // FB

            def _fl(q, _):
                _flush(q, jnp.int32(FB))
                return 0

            lax.fori_loop(0, nb, _fl, 0)

            def _cd():
                rem0 = nb * FB

                def _mv(i, _):
                    sl_src = pl.ds(rem0 + i * LANES, LANES)
                    sl_dst = pl.ds(i * LANES, LANES)
                    srcsel_v[sl_dst] = srcsel_v[sl_src]
                    dlocsel_v[sl_dst] = dlocsel_v[sl_src]
                    eidsel_v[sl_dst] = eidsel_v[sl_src]
                    return 0

                lax.fori_loop(0, FB // LANES, _mv, 0)

            pl.when(nb > 0)(_cd)
            return cnt - nb * FB

        cnt = lax.fori_loop(0, NBLK, _blk, jnp.int32(0))

        def _tail():
            _flush(jnp.int32(0), cnt)

        pl.when(cnt > 0)(_tail)

        pltpu.sync_copy(acc_v.at[pl.ds(0, W * HID)],
                        out_hbm.at[pl.ds(lo * HID, W * HID)])


def _prep_edges(edge_index):
    src = edge_index[0]
    dst = edge_index[1]
    pad = EPAD - E
    src_p = jnp.concatenate([src, jnp.zeros((pad,), jnp.int32)])
    dst_p = jnp.concatenate([dst, jnp.full((pad,), SENT, jnp.int32)])
    return src_p, dst_p


def _gat_layer_a(h, p, src, dst, src_p, dst_p):
    n = h.shape[0]
    hp = (h @ p['W']).reshape(n, HEADS, C)
    e_src = jnp.sum(hp * p['a_src'], axis=-1)
    e_dst = jnp.sum(hp * p['a_dst'], axis=-1)
    es_T = jnp.zeros((HEADS, NPAD), jnp.float32).at[:, :N].set(e_src.T).reshape(-1)
    ed_T = jnp.zeros((HEADS, NPAD), jnp.float32).at[:, :N].set(e_dst.T).reshape(-1)
    den_parts, ex_T = _sc_den(es_T, ed_T, src_p, dst_p)
    out = _sc_agg(hp.reshape(n, HID), ex_T, den_parts, src_p, dst_p)
    return out.reshape(NOUT, HID)[:N] + p['b']


def kernel(x, params, edge_index):
    src = edge_index[0]
    dst = edge_index[1]
    src_p, dst_p = _prep_edges(edge_index)
    h = x
    ngat = len(params['gat'])
    for i, p in enumerate(params['gat']):
        h = _gat_layer_a(h, p, src, dst, src_p, dst_p)
        if i < ngat - 1:
            h = jax.nn.elu(h)
    g = jax.nn.gelu(h @ params['proj_W1'] + params['proj_b1']) @ params['proj_W2'] + params['proj_b2']
    go = g[None, :, :]
    q = params['queries'][None, :, :]
    Q = (q @ params['Wq'] + params['bq']).reshape(1, NQ, AH, AC).transpose(0, 2, 1, 3)
    K = (go @ params['Wk'] + params['bk']).reshape(1, N, AH, AC).transpose(0, 2, 1, 3)
    V = (go @ params['Wv'] + params['bv']).reshape(1, N, AH, AC).transpose(0, 2, 1, 3)
    scores = jnp.einsum('bhqd,bhkd->bhqk', Q, K) / float(np.sqrt(AC))
    attn = jax.nn.softmax(scores, axis=-1)
    ctx = jnp.einsum('bhqk,bhkd->bhqd', attn, V).transpose(0, 2, 1, 3).reshape(1, NQ, D_IN)
    ao = ctx @ params['Wo'] + params['bo']

    def _ln(t, g_, b_):
        m = jnp.mean(t, axis=-1, keepdims=True)
        v = jnp.var(t, axis=-1, keepdims=True)
        return (t - m) / jnp.sqrt(v + 1e-5) * g_ + b_

    h1 = _ln(q + ao, params['ln1_g'], params['ln1_b'])
    f = jax.nn.gelu(h1 @ params['ffn_W1'] + params['ffn_b1']) @ params['ffn_W2'] + params['ffn_b2']
    return _ln(h1 + f, params['ln2_g'], params['ln2_b'])


# speedup vs baseline: 9.9800x; 1.0848x over previous
import functools

import jax
import jax.numpy as jnp
import numpy as np
from jax import lax
from jax.experimental import pallas as pl
from jax.experimental.pallas import tpu as pltpu
from jax.experimental.pallas import tpu_sc as plsc

N = 10000
E = 160000
D_IN = 256
HID = 512
HEADS = 8
C = HID // HEADS
NQ = 200
AH = 8
AC = D_IN // AH
NEG_SLOPE = 0.2

NC = 2
NS = 16
LANES = 16
NW = NC * NS

NPAD = 12544
SENT = 10240
EPAD = 163840
ESH = EPAD // NW
GRP = ESH // LANES
STRIPE = NPAD // NS

_mesh = plsc.VectorSubcoreMesh(core_axis_name="c", subcore_axis_name="s")
_sc_params = pltpu.CompilerParams(needs_layout_passes=False)


@functools.partial(
    pl.kernel,
    out_type=[
        jax.ShapeDtypeStruct((NC * HEADS * NPAD,), jnp.float32),
        jax.ShapeDtypeStruct((HEADS * EPAD,), jnp.float32),
    ],
    mesh=_mesh,
    compiler_params=_sc_params,
    scratch_types=[
        pltpu.VMEM((ESH,), jnp.int32),
        pltpu.VMEM((ESH,), jnp.int32),
        pltpu.VMEM((NPAD,), jnp.float32),
        pltpu.VMEM((NPAD,), jnp.float32),
        pltpu.VMEM((NPAD,), jnp.float32),
        pltpu.VMEM((ESH,), jnp.float32),
        pltpu.VMEM((STRIPE,), jnp.float32),
        pltpu.VMEM((STRIPE,), jnp.float32),
        pltpu.VMEM_SHARED((NS * NPAD,), jnp.float32),
    ],
)
def _sc_den(es_hbm, ed_hbm, src_hbm, dst_hbm, den_hbm, ex_hbm,
            src_v, dst_v, es_v, ed_v, den_v, ex_v, tmp_v, acc_v, slots):
    c = lax.axis_index("c")
    s = lax.axis_index("s")
    w = 2 * s + c
    e0 = w * ESH
    pltpu.sync_copy(src_hbm.at[pl.ds(e0, ESH)], src_v)
    pltpu.sync_copy(dst_hbm.at[pl.ds(e0, ESH)], dst_v)
    zero16 = jnp.zeros((LANES,), jnp.float32)
    for h in range(HEADS):
        pltpu.sync_copy(es_hbm.at[pl.ds(h * NPAD, NPAD)], es_v)
        pltpu.sync_copy(ed_hbm.at[pl.ds(h * NPAD, NPAD)], ed_v)

        def _zero(i, _):
            den_v[pl.ds(i * LANES, LANES)] = zero16
            return 0

        lax.fori_loop(0, NPAD // LANES, _zero, 0)

        def _edge(i, _):
            si = src_v[pl.ds(i * LANES, LANES)]
            di = dst_v[pl.ds(i * LANES, LANES)]
            a = plsc.load_gather(es_v, [si])
            b = plsc.load_gather(ed_v, [di])
            e = a + b
            e = jnp.where(e > 0, e, NEG_SLOPE * e)
            ex = jnp.exp(e)
            ex_v[pl.ds(i * LANES, LANES)] = ex
            plsc.addupdate_scatter(den_v, [di], ex)
            return 0

        lax.fori_loop(0, GRP, _edge, 0)
        pltpu.sync_copy(ex_v, ex_hbm.at[pl.ds(h * EPAD + e0, ESH)])

        pltpu.sync_copy(den_v, slots.at[pl.ds(s * NPAD, NPAD)])
        plsc.subcore_barrier()
        base = s * STRIPE

        def _zs(i, _):
            acc_v[pl.ds(i * LANES, LANES)] = zero16
            return 0

        lax.fori_loop(0, STRIPE // LANES, _zs, 0)
        for j in range(NS):
            pltpu.sync_copy(slots.at[pl.ds(j * NPAD + base, STRIPE)], tmp_v)

            def _acc(i, _):
                sl = pl.ds(i * LANES, LANES)
                acc_v[sl] = acc_v[sl] + tmp_v[sl]
                return 0

            lax.fori_loop(0, STRIPE // LANES, _acc, 0)
        pltpu.sync_copy(acc_v, den_hbm.at[pl.ds((c * HEADS + h) * NPAD + base, STRIPE)])
        plsc.subcore_barrier()


W = 160
NRND = 2
NOUT = NRND * NW * W
GBLK = 2048
NBLK = EPAD // GBLK
SEL = GBLK + 64
FB = 64
ACCW = W * HID


@functools.partial(
    pl.kernel,
    out_type=jax.ShapeDtypeStruct((NOUT * HID,), jnp.float32),
    mesh=_mesh,
    compiler_params=_sc_params,
    scratch_types=[
        pltpu.VMEM((2 * GBLK,), jnp.int32),
        pltpu.VMEM((SEL,), jnp.int32),
        pltpu.VMEM((SEL,), jnp.int32),
        pltpu.VMEM((SEL,), jnp.int32),
        pltpu.VMEM((HEADS * FB,), jnp.int32),
        pltpu.VMEM((HEADS * FB,), jnp.float32),
        pltpu.VMEM((HEADS * FB,), jnp.float32),
        pltpu.VMEM((HEADS * W,), jnp.float32),
        pltpu.VMEM((W,), jnp.float32),
        pltpu.VMEM((W,), jnp.float32),
        pltpu.VMEM((FB, HID), jnp.float32),
        pltpu.VMEM((ACCW,), jnp.float32),
    ],
)
def _sc_agg(hp_hbm, ex_hbm, den_hbm, blk_hbm, out_hbm,
            blk_v, srcsel_v, dlocsel_v, eidsel_v, idxh_v,
            exb_v, alpha_v, invd_v, da_v, db_v, rows_v, acc_v):
    c = lax.axis_index("c")
    s = lax.axis_index("s")
    t = 2 * s + c
    zero16 = jnp.zeros((LANES,), jnp.float32)
    iota16 = lax.iota(jnp.int32, LANES)

    def _zi(i, _):
        sl = pl.ds(i * LANES, LANES)
        z = jnp.zeros((LANES,), jnp.int32)
        srcsel_v[sl] = z
        dlocsel_v[sl] = z
        eidsel_v[sl] = z
        return 0

    lax.fori_loop(0, SEL // LANES, _zi, 0)

    def _flush(q, vc):
        def _ix(i, _):
            e16 = eidsel_v[pl.ds(q * FB + i * LANES, LANES)]
            for h in range(HEADS):
                idxh_v[pl.ds(h * FB + i * LANES, LANES)] = e16 + h * EPAD
            return 0

        lax.fori_loop(0, FB // LANES, _ix, 0)
        pltpu.sync_copy(ex_hbm.at[idxh_v], exb_v)
        def _al(i, _):
            lane = iota16 + i * LANES
            valid = lane < vc
            d16 = dlocsel_v[pl.ds(q * FB + i * LANES, LANES)]
            gidx = jnp.where(valid, d16, 0)
            for h in range(HEADS):
                ivd = plsc.load_gather(invd_v, [gidx + h * W])
                exv = exb_v[pl.ds(h * FB + i * LANES, LANES)]
                alpha_v[pl.ds(h * FB + i * LANES, LANES)] = \
                    jnp.where(valid, exv * ivd, 0.0)
            return 0

        lax.fori_loop(0, FB // LANES, _al, 0)
        pltpu.sync_copy(hp_hbm.at[srcsel_v.at[pl.ds(q * FB, FB)]], rows_v)

        def _edge(e, _):
            db = plsc.load_gather(dlocsel_v, [jnp.broadcast_to(q * FB + e, (LANES,))])
            addr0 = db * HID + iota16
            for h in range(HEADS):
                av = plsc.load_gather(alpha_v, [jnp.broadcast_to(h * FB + e, (LANES,))])
                for qq in range(4):
                    off = h * C + qq * LANES
                    chunk = rows_v[e, pl.ds(off, LANES)] * av
                    plsc.addupdate_scatter(acc_v, [addr0 + off], chunk)
            return 0

        lax.fori_loop(0, FB, _edge, 0)

    for r in range(NRND):
        lo = (r * NW + t) * W
        for h in range(HEADS):
            pltpu.sync_copy(den_hbm.at[pl.ds(h * NPAD + lo, W)], da_v)
            pltpu.sync_copy(den_hbm.at[pl.ds(HEADS * NPAD + h * NPAD + lo, W)], db_v)

            def _inv(i, _):
                sl = pl.ds(i * LANES, LANES)
                invd_v[pl.ds(h * W + i * LANES, LANES)] = \
                    1.0 / (da_v[sl] + db_v[sl] + 1e-16)
                return 0

            lax.fori_loop(0, W // LANES, _inv, 0)

        def _za(i, _):
            acc_v[pl.ds(i * LANES, LANES)] = zero16
            return 0

        lax.fori_loop(0, ACCW // LANES, _za, 0)

        def _blk(bi, cnt):
            off = bi * GBLK
            pltpu.sync_copy(blk_hbm.at[pl.ds(2 * off, 2 * GBLK)], blk_v)

            def _grp(i, cn):
                d16 = blk_v[pl.ds(GBLK + i * LANES, LANES)]
                in_w = (d16 >= lo) & (d16 < lo + W)
                pc = plsc.all_reduce_population_count(in_w)[0]

                def _stage(cn2):
                    s16 = blk_v[pl.ds(i * LANES, LANES)]
                    eid = off + i * LANES + iota16
                    plsc.store_compressed(srcsel_v.at[pl.ds(cn2, LANES)], s16, mask=in_w)
                    plsc.store_compressed(dlocsel_v.at[pl.ds(cn2, LANES)], d16 - lo, mask=in_w)
                    plsc.store_compressed(eidsel_v.at[pl.ds(cn2, LANES)], eid, mask=in_w)
                    return cn2 + pc

                return lax.cond(pc > 0, _stage, lambda x: x, cn)

            cnt = lax.fori_loop(0, GBLK // LANES, _grp, cnt)
            nb = cnt // FB

            def _fl(q, _):
                _flush(q, jnp.int32(FB))
                return 0

            lax.fori_loop(0, nb, _fl, 0)

            def _cd():
                rem0 = nb * FB

                def _mv(i, _):
                    sl_src = pl.ds(rem0 + i * LANES, LANES)
                    sl_dst = pl.ds(i * LANES, LANES)
                    srcsel_v[sl_dst] = srcsel_v[sl_src]
                    dlocsel_v[sl_dst] = dlocsel_v[sl_src]
                    eidsel_v[sl_dst] = eidsel_v[sl_src]
                    return 0

                lax.fori_loop(0, FB // LANES, _mv, 0)

            pl.when(nb > 0)(_cd)
            return cnt - nb * FB

        cnt = lax.fori_loop(0, NBLK, _blk, jnp.int32(0))

        def _tail():
            _flush(jnp.int32(0), cnt)

        pl.when(cnt > 0)(_tail)

        pltpu.sync_copy(acc_v.at[pl.ds(0, W * HID)],
                        out_hbm.at[pl.ds(lo * HID, W * HID)])


def _prep_edges(edge_index):
    src = edge_index[0]
    dst = edge_index[1]
    pad = EPAD - E
    src_p = jnp.concatenate([src, jnp.zeros((pad,), jnp.int32)])
    dst_p = jnp.concatenate([dst, jnp.full((pad,), SENT, jnp.int32)])
    blk = jnp.stack([src_p.reshape(NBLK, GBLK),
                     dst_p.reshape(NBLK, GBLK)], axis=1).reshape(-1)
    return src_p, dst_p, blk


def _gat_layer_a(h, p, src, dst, src_p, dst_p, blk):
    n = h.shape[0]
    hp = (h @ p['W']).reshape(n, HEADS, C)
    e_src = jnp.sum(hp * p['a_src'], axis=-1)
    e_dst = jnp.sum(hp * p['a_dst'], axis=-1)
    es_T = jnp.zeros((HEADS, NPAD), jnp.float32).at[:, :N].set(e_src.T).reshape(-1)
    ed_T = jnp.zeros((HEADS, NPAD), jnp.float32).at[:, :N].set(e_dst.T).reshape(-1)
    den_parts, ex_T = _sc_den(es_T, ed_T, src_p, dst_p)
    out = _sc_agg(hp.reshape(n, HID), ex_T, den_parts, blk)
    return out.reshape(NOUT, HID)[:N] + p['b']


def kernel(x, params, edge_index):
    src = edge_index[0]
    dst = edge_index[1]
    src_p, dst_p, blk = _prep_edges(edge_index)
    h = x
    ngat = len(params['gat'])
    for i, p in enumerate(params['gat']):
        h = _gat_layer_a(h, p, src, dst, src_p, dst_p, blk)
        if i < ngat - 1:
            h = jax.nn.elu(h)
    g = jax.nn.gelu(h @ params['proj_W1'] + params['proj_b1']) @ params['proj_W2'] + params['proj_b2']
    go = g[None, :, :]
    q = params['queries'][None, :, :]
    Q = (q @ params['Wq'] + params['bq']).reshape(1, NQ, AH, AC).transpose(0, 2, 1, 3)
    K = (go @ params['Wk'] + params['bk']).reshape(1, N, AH, AC).transpose(0, 2, 1, 3)
    V = (go @ params['Wv'] + params['bv']).reshape(1, N, AH, AC).transpose(0, 2, 1, 3)
    scores = jnp.einsum('bhqd,bhkd->bhqk', Q, K) / float(np.sqrt(AC))
    attn = jax.nn.softmax(scores, axis=-1)
    ctx = jnp.einsum('bhqk,bhkd->bhqd', attn, V).transpose(0, 2, 1, 3).reshape(1, NQ, D_IN)
    ao = ctx @ params['Wo'] + params['bo']

    def _ln(t, g_, b_):
        m = jnp.mean(t, axis=-1, keepdims=True)
        v = jnp.var(t, axis=-1, keepdims=True)
        return (t - m) / jnp.sqrt(v + 1e-5) * g_ + b_

    h1 = _ln(q + ao, params['ln1_g'], params['ln1_b'])
    f = jax.nn.gelu(h1 @ params['ffn_W1'] + params['ffn_b1']) @ params['ffn_W2'] + params['ffn_b2']
    return _ln(h1 + f, params['ln2_g'], params['ln2_b'])


# speedup vs baseline: 11.4547x; 1.1478x over previous
import functools

import jax
import jax.numpy as jnp
import numpy as np
from jax import lax
from jax.experimental import pallas as pl
from jax.experimental.pallas import tpu as pltpu
from jax.experimental.pallas import tpu_sc as plsc

N = 10000
E = 160000
D_IN = 256
HID = 512
HEADS = 8
C = HID // HEADS
NQ = 200
AH = 8
AC = D_IN // AH
NEG_SLOPE = 0.2

NC = 2
NS = 16
LANES = 16
NW = NC * NS

NPAD = 12544
SENT = 10240
EPAD = 163840
ESH = EPAD // NW
GRP = ESH // LANES
STRIPE = NPAD // NS

_mesh = plsc.VectorSubcoreMesh(core_axis_name="c", subcore_axis_name="s")
_sc_params = pltpu.CompilerParams(needs_layout_passes=False)


@functools.partial(
    pl.kernel,
    out_type=[
        jax.ShapeDtypeStruct((NC * HEADS * NPAD,), jnp.float32),
        jax.ShapeDtypeStruct((HEADS * EPAD,), jnp.float32),
    ],
    mesh=_mesh,
    compiler_params=_sc_params,
    scratch_types=[
        pltpu.VMEM((ESH,), jnp.int32),
        pltpu.VMEM((ESH,), jnp.int32),
        pltpu.VMEM((NPAD,), jnp.float32),
        pltpu.VMEM((NPAD,), jnp.float32),
        pltpu.VMEM((NPAD,), jnp.float32),
        pltpu.VMEM((ESH,), jnp.float32),
        pltpu.VMEM((STRIPE,), jnp.float32),
        pltpu.VMEM((STRIPE,), jnp.float32),
        pltpu.VMEM_SHARED((NS * NPAD,), jnp.float32),
    ],
)
def _sc_den(es_hbm, ed_hbm, src_hbm, dst_hbm, den_hbm, ex_hbm,
            src_v, dst_v, es_v, ed_v, den_v, ex_v, tmp_v, acc_v, slots):
    c = lax.axis_index("c")
    s = lax.axis_index("s")
    w = 2 * s + c
    e0 = w * ESH
    pltpu.sync_copy(src_hbm.at[pl.ds(e0, ESH)], src_v)
    pltpu.sync_copy(dst_hbm.at[pl.ds(e0, ESH)], dst_v)
    zero16 = jnp.zeros((LANES,), jnp.float32)
    for h in range(HEADS):
        pltpu.sync_copy(es_hbm.at[pl.ds(h * NPAD, NPAD)], es_v)
        pltpu.sync_copy(ed_hbm.at[pl.ds(h * NPAD, NPAD)], ed_v)

        def _zero(i, _):
            den_v[pl.ds(i * LANES, LANES)] = zero16
            return 0

        lax.fori_loop(0, NPAD // LANES, _zero, 0)

        def _edge(i, _):
            si = src_v[pl.ds(i * LANES, LANES)]
            di = dst_v[pl.ds(i * LANES, LANES)]
            a = plsc.load_gather(es_v, [si])
            b = plsc.load_gather(ed_v, [di])
            e = a + b
            e = jnp.where(e > 0, e, NEG_SLOPE * e)
            ex = jnp.exp(e)
            ex_v[pl.ds(i * LANES, LANES)] = ex
            plsc.addupdate_scatter(den_v, [di], ex)
            return 0

        lax.fori_loop(0, GRP, _edge, 0)
        pltpu.sync_copy(ex_v, ex_hbm.at[pl.ds(h * EPAD + e0, ESH)])

        pltpu.sync_copy(den_v, slots.at[pl.ds(s * NPAD, NPAD)])
        plsc.subcore_barrier()
        base = s * STRIPE

        def _zs(i, _):
            acc_v[pl.ds(i * LANES, LANES)] = zero16
            return 0

        lax.fori_loop(0, STRIPE // LANES, _zs, 0)
        for j in range(NS):
            pltpu.sync_copy(slots.at[pl.ds(j * NPAD + base, STRIPE)], tmp_v)

            def _acc(i, _):
                sl = pl.ds(i * LANES, LANES)
                acc_v[sl] = acc_v[sl] + tmp_v[sl]
                return 0

            lax.fori_loop(0, STRIPE // LANES, _acc, 0)
        pltpu.sync_copy(acc_v, den_hbm.at[pl.ds((c * HEADS + h) * NPAD + base, STRIPE)])
        plsc.subcore_barrier()


W = 160
NRND = 2
NOUT = NRND * NW * W
GBLK = 2048
NBLK = EPAD // GBLK
FB = 64
ACCW = W * HID
CH = 1024
NCH = EPAD // CH
SCAP = 3072
STRD = EPAD + 4096


@functools.partial(
    pl.kernel,
    out_type=[
        jax.ShapeDtypeStruct((NW * 2 * STRD,), jnp.int32),
        jax.ShapeDtypeStruct((NW * LANES,), jnp.int32),
    ],
    mesh=_mesh,
    compiler_params=_sc_params,
    scratch_types=[
        pltpu.VMEM((2 * GBLK,), jnp.int32),
        pltpu.VMEM((SCAP,), jnp.int32),
        pltpu.VMEM((SCAP,), jnp.int32),
        pltpu.VMEM((LANES,), jnp.int32),
    ],
)
def _sc_bin(blk_hbm, spill_hbm, cnts_hbm, blk_v, e0_v, e1_v, cw_v):
    c = lax.axis_index("c")
    s = lax.axis_index("s")
    t = 2 * s + c
    iota16 = lax.iota(jnp.int32, LANES)
    lo0 = t * W
    lo1 = (NW + t) * W
    b0 = (t * 2) * STRD
    b1 = (t * 2 + 1) * STRD

    def _zi(i, _):
        z = jnp.zeros((LANES,), jnp.int32)
        e0_v[pl.ds(i * LANES, LANES)] = z
        e1_v[pl.ds(i * LANES, LANES)] = z
        return 0

    lax.fori_loop(0, SCAP // LANES, _zi, 0)

    def _blk(bi, st):
        c0, c1, o0, o1 = st
        off = bi * GBLK
        pltpu.sync_copy(blk_hbm.at[pl.ds(2 * off, 2 * GBLK)], blk_v)

        def _grp(i, cc):
            cn0, cn1 = cc
            d16 = blk_v[pl.ds(GBLK + i * LANES, LANES)]
            eid = off + i * LANES + iota16
            m0 = (d16 >= lo0) & (d16 < lo0 + W)
            m1 = (d16 >= lo1) & (d16 < lo1 + W)
            p0 = plsc.all_reduce_population_count(m0)[0]
            p1 = plsc.all_reduce_population_count(m1)[0]

            def _s0(x):
                plsc.store_compressed(e0_v.at[pl.ds(x, LANES)], eid, mask=m0)
                return x + p0

            def _s1(x):
                plsc.store_compressed(e1_v.at[pl.ds(x, LANES)], eid, mask=m1)
                return x + p1

            cn0 = lax.cond(p0 > 0, _s0, lambda x: x, cn0)
            cn1 = lax.cond(p1 > 0, _s1, lambda x: x, cn1)
            return (cn0, cn1)

        c0, c1 = lax.fori_loop(0, GBLK // LANES, _grp, (c0, c1))

        def _flw(ebuf, base):
            def _do(st2):
                cn, wo = st2
                pltpu.sync_copy(ebuf.at[pl.ds(0, 1024)],
                                spill_hbm.at[pl.ds(base + wo * 1024, 1024)])

                def _sh(i, _):
                    ebuf[pl.ds(i * LANES, LANES)] = \
                        ebuf[pl.ds(1024 + i * LANES, LANES)]
                    return 0

                lax.fori_loop(0, 2048 // LANES, _sh, 0)
                return (cn - 1024, wo + 1)

            return _do

        for _ in range(2):
            c0, o0 = lax.cond(c0 >= 1024, _flw(e0_v, b0), lambda x: x, (c0, o0))
            c1, o1 = lax.cond(c1 >= 1024, _flw(e1_v, b1), lambda x: x, (c1, o1))
        return (c0, c1, o0, o1)

    z = jnp.int32(0)
    c0, c1, o0, o1 = lax.fori_loop(0, NBLK, _blk, (z, z, z, z))
    pltpu.sync_copy(e0_v, spill_hbm.at[pl.ds(b0 + o0 * 1024, SCAP)])
    pltpu.sync_copy(e1_v, spill_hbm.at[pl.ds(b1 + o1 * 1024, SCAP)])
    t0 = o0 * 1024 + c0
    t1 = o1 * 1024 + c1
    cw_v[pl.ds(0, LANES)] = jnp.where(
        iota16 == 0, t0, jnp.where(iota16 == 1, t1, 0))
    pltpu.sync_copy(cw_v, cnts_hbm.at[pl.ds(t * LANES, LANES)])


@functools.partial(
    pl.kernel,
    out_type=jax.ShapeDtypeStruct((NOUT * HID,), jnp.float32),
    mesh=_mesh,
    compiler_params=_sc_params,
    scratch_types=[
        pltpu.VMEM((CH,), jnp.int32),
        pltpu.VMEM((CH,), jnp.int32),
        pltpu.VMEM((CH,), jnp.int32),
        pltpu.VMEM((CH,), jnp.int32),
        pltpu.VMEM((LANES,), jnp.int32),
        pltpu.VMEM((HEADS * FB,), jnp.int32),
        pltpu.VMEM((HEADS * FB,), jnp.float32),
        pltpu.VMEM((HEADS * FB,), jnp.float32),
        pltpu.VMEM((HEADS * W,), jnp.float32),
        pltpu.VMEM((W,), jnp.float32),
        pltpu.VMEM((W,), jnp.float32),
        pltpu.VMEM((FB, HID), jnp.float32),
        pltpu.VMEM((ACCW,), jnp.float32),
    ],
)
def _sc_agg(hp_hbm, ex_hbm, den_hbm, blk_hbm, spill_hbm, cnts_hbm, out_hbm,
            srcsel_v, dlocsel_v, eidsel_v, tmpidx_v, cnt_v, idxh_v,
            exb_v, alpha_v, invd_v, da_v, db_v, rows_v, acc_v):
    c = lax.axis_index("c")
    s = lax.axis_index("s")
    t = 2 * s + c
    zero16 = jnp.zeros((LANES,), jnp.float32)
    iota16 = lax.iota(jnp.int32, LANES)
    pltpu.sync_copy(cnts_hbm.at[pl.ds(t * LANES, LANES)], cnt_v)
    cnts = cnt_v[pl.ds(0, LANES)]

    def _flush(q, vc):
        def _ix(i, _):
            e16 = eidsel_v[pl.ds(q * FB + i * LANES, LANES)]
            for h in range(HEADS):
                idxh_v[pl.ds(h * FB + i * LANES, LANES)] = e16 + h * EPAD
            return 0

        lax.fori_loop(0, FB // LANES, _ix, 0)
        pltpu.sync_copy(ex_hbm.at[idxh_v], exb_v)
        def _al(i, _):
            lane = iota16 + i * LANES
            valid = lane < vc
            d16 = dlocsel_v[pl.ds(q * FB + i * LANES, LANES)]
            gidx = jnp.where(valid, d16, 0)
            for h in range(HEADS):
                ivd = plsc.load_gather(invd_v, [gidx + h * W])
                exv = exb_v[pl.ds(h * FB + i * LANES, LANES)]
                alpha_v[pl.ds(h * FB + i * LANES, LANES)] = \
                    jnp.where(valid, exv * ivd, 0.0)
            return 0

        lax.fori_loop(0, FB // LANES, _al, 0)
        pltpu.sync_copy(hp_hbm.at[srcsel_v.at[pl.ds(q * FB, FB)]], rows_v)

        def _edge(e, _):
            db = plsc.load_gather(dlocsel_v, [jnp.broadcast_to(q * FB + e, (LANES,))])
            addr0 = db * HID + iota16
            for h in range(HEADS):
                av = plsc.load_gather(alpha_v, [jnp.broadcast_to(h * FB + e, (LANES,))])
                for qq in range(4):
                    off = h * C + qq * LANES
                    chunk = rows_v[e, pl.ds(off, LANES)] * av
                    plsc.addupdate_scatter(acc_v, [addr0 + off], chunk)
            return 0

        lax.fori_loop(0, FB, _edge, 0)

    for r in range(NRND):
        lo = (r * NW + t) * W
        for h in range(HEADS):
            pltpu.sync_copy(den_hbm.at[pl.ds(h * NPAD + lo, W)], da_v)
            pltpu.sync_copy(den_hbm.at[pl.ds(HEADS * NPAD + h * NPAD + lo, W)], db_v)

            def _inv(i, _):
                sl = pl.ds(i * LANES, LANES)
                invd_v[pl.ds(h * W + i * LANES, LANES)] = \
                    1.0 / (da_v[sl] + db_v[sl] + 1e-16)
                return 0

            lax.fori_loop(0, W // LANES, _inv, 0)

        def _za(i, _):
            acc_v[pl.ds(i * LANES, LANES)] = zero16
            return 0

        lax.fori_loop(0, ACCW // LANES, _za, 0)

        cnt = cnts[r]
        sbase = (t * 2 + r) * STRD

        def _chunk(b):
            pltpu.sync_copy(spill_hbm.at[pl.ds(sbase + b * CH, CH)], eidsel_v)

            def _p1(i, _):
                sl = pl.ds(i * LANES, LANES)
                lane = b * CH + i * LANES + iota16
                e16 = jnp.where(lane < cnt, eidsel_v[sl], 0)
                eidsel_v[sl] = e16
                tmpidx_v[sl] = e16 + (e16 // GBLK) * GBLK
                return 0

            lax.fori_loop(0, CH // LANES, _p1, 0)
            pltpu.sync_copy(blk_hbm.at[tmpidx_v], srcsel_v)

            def _p2(i, _):
                sl = pl.ds(i * LANES, LANES)
                tmpidx_v[sl] = tmpidx_v[sl] + GBLK
                return 0

            lax.fori_loop(0, CH // LANES, _p2, 0)
            pltpu.sync_copy(blk_hbm.at[tmpidx_v], dlocsel_v)

            def _p3(i, _):
                sl = pl.ds(i * LANES, LANES)
                lane = b * CH + i * LANES + iota16
                dloc = dlocsel_v[sl] - lo
                dlocsel_v[sl] = jnp.where(lane < cnt, dloc, 0)
                return 0

            lax.fori_loop(0, CH // LANES, _p3, 0)
            def _fb(q, _):
                vc = jnp.clip(cnt - (b * CH + q * FB), 0, FB)
                pl.when(vc > 0)(lambda: _flush(q, vc))
                return 0

            lax.fori_loop(0, CH // FB, _fb, 0)

        def _chk(b, _):
            pl.when(b * CH < cnt)(lambda: _chunk(b))
            return 0

        lax.fori_loop(0, NCH, _chk, 0)

        pltpu.sync_copy(acc_v.at[pl.ds(0, W * HID)],
                        out_hbm.at[pl.ds(lo * HID, W * HID)])


def _prep_edges(edge_index):
    src = edge_index[0]
    dst = edge_index[1]
    pad = EPAD - E
    src_p = jnp.concatenate([src, jnp.zeros((pad,), jnp.int32)])
    dst_p = jnp.concatenate([dst, jnp.full((pad,), SENT, jnp.int32)])
    blk = jnp.stack([src_p.reshape(NBLK, GBLK),
                     dst_p.reshape(NBLK, GBLK)], axis=1).reshape(-1)
    return src_p, dst_p, blk


def _gat_layer_a(h, p, src_p, dst_p, blk, spill, cnts):
    n = h.shape[0]
    hp = (h @ p['W']).reshape(n, HEADS, C)
    e_src = jnp.sum(hp * p['a_src'], axis=-1)
    e_dst = jnp.sum(hp * p['a_dst'], axis=-1)
    es_T = jnp.zeros((HEADS, NPAD), jnp.float32).at[:, :N].set(e_src.T).reshape(-1)
    ed_T = jnp.zeros((HEADS, NPAD), jnp.float32).at[:, :N].set(e_dst.T).reshape(-1)
    den_parts, ex_T = _sc_den(es_T, ed_T, src_p, dst_p)
    out = _sc_agg(hp.reshape(n, HID), ex_T, den_parts, blk, spill, cnts)
    return out.reshape(NOUT, HID)[:N] + p['b']


def kernel(x, params, edge_index):
    src = edge_index[0]
    dst = edge_index[1]
    src_p, dst_p, blk = _prep_edges(edge_index)
    spill, cnts = _sc_bin(blk)
    h = x
    ngat = len(params['gat'])
    for i, p in enumerate(params['gat']):
        h = _gat_layer_a(h, p, src_p, dst_p, blk, spill, cnts)
        if i < ngat - 1:
            h = jax.nn.elu(h)
    g = jax.nn.gelu(h @ params['proj_W1'] + params['proj_b1']) @ params['proj_W2'] + params['proj_b2']
    go = g[None, :, :]
    q = params['queries'][None, :, :]
    Q = (q @ params['Wq'] + params['bq']).reshape(1, NQ, AH, AC).transpose(0, 2, 1, 3)
    K = (go @ params['Wk'] + params['bk']).reshape(1, N, AH, AC).transpose(0, 2, 1, 3)
    V = (go @ params['Wv'] + params['bv']).reshape(1, N, AH, AC).transpose(0, 2, 1, 3)
    scores = jnp.einsum('bhqd,bhkd->bhqk', Q, K) / float(np.sqrt(AC))
    attn = jax.nn.softmax(scores, axis=-1)
    ctx = jnp.einsum('bhqk,bhkd->bhqd', attn, V).transpose(0, 2, 1, 3).reshape(1, NQ, D_IN)
    ao = ctx @ params['Wo'] + params['bo']

    def _ln(t, g_, b_):
        m = jnp.mean(t, axis=-1, keepdims=True)
        v = jnp.var(t, axis=-1, keepdims=True)
        return (t - m) / jnp.sqrt(v + 1e-5) * g_ + b_

    h1 = _ln(q + ao, params['ln1_g'], params['ln1_b'])
    f = jax.nn.gelu(h1 @ params['ffn_W1'] + params['ffn_b1']) @ params['ffn_W2'] + params['ffn_b2']
    return _ln(h1 + f, params['ln2_g'], params['ln2_b'])
